# baseline (speedup 1.0000x reference)
import functools

import jax
import jax.numpy as jnp
from jax import lax
from jax.experimental import pallas as pl
from jax.experimental.pallas import tpu as pltpu
from jax.experimental.pallas import tpu_sc as plsc

N = 10000
DEG = 16
D_IN = 256
D_OUT = 256

LANES = 16
NC, NS = 2, 16
NW = NC * NS
BLK = 8
NBLK = N // BLK
DJ = D_IN // LANES

A_SPLIT = 7000
B0 = A_SPLIT // BLK


def _sc_sum_body(neib_hbm, out_hbm, buf, acc, sem_in0, sem_in1, sem_out0,
                 sem_out1):
    wid = lax.axis_index("s") * NC + lax.axis_index("c")
    nt = (NBLK - B0 - 1 - wid) // NW + 1
    sems_in = (sem_in0, sem_in1)
    sems_out = (sem_out0, sem_out1)

    def start_in(t, slot):
        blk = (B0 + wid + t * NW) * BLK
        pltpu.async_copy(neib_hbm.at[pl.ds(blk, BLK)], buf.at[slot],
                         sems_in[slot])

    @pl.when(nt > 0)
    def _prime():
        start_in(0, 0)

    @pl.loop(0, nt, step=2)
    def _pair(t0):
        for p in range(2):
            t = t0 + p

            @pl.when(t < nt)
            def _one():
                pltpu.make_async_copy(neib_hbm.at[pl.ds(0, BLK)], buf.at[p],
                                      sems_in[p]).wait()

                @pl.when(t + 1 < nt)
                def _next():
                    start_in(t + 1, 1 - p)

                @pl.when(t >= 2)
                def _drain():
                    pltpu.make_async_copy(acc.at[p],
                                          out_hbm.at[pl.ds(0, BLK)],
                                          sems_out[p]).wait()

                @pl.loop(0, BLK)
                def _node(i):
                    for j in range(DJ):
                        vs = [buf[p, i, k, pl.ds(j * LANES, LANES)]
                              for k in range(DEG)]
                        while len(vs) > 1:
                            vs = [vs[a] + vs[a + 1]
                                  for a in range(0, len(vs) - 1, 2)] + (
                                      [vs[-1]] if len(vs) % 2 else [])
                        acc[p, i, pl.ds(j * LANES, LANES)] = vs[0]

                pltpu.async_copy(
                    acc.at[p],
                    out_hbm.at[pl.ds((wid + t * NW) * BLK, BLK)],
                    sems_out[p])

    parity = (nt - 1) % 2
    for p in range(2):
        @pl.when(jnp.logical_and(nt >= 1, parity == p))
        def _last():
            pltpu.make_async_copy(acc.at[p], out_hbm.at[pl.ds(0, BLK)],
                                  sems_out[p]).wait()

        @pl.when(jnp.logical_and(nt >= 2, parity == 1 - p))
        def _second_last():
            pltpu.make_async_copy(acc.at[p], out_hbm.at[pl.ds(0, BLK)],
                                  sems_out[p]).wait()


@functools.cache
def _make_sc_neib_sum():
    return pl.kernel(
        _sc_sum_body,
        out_type=jax.ShapeDtypeStruct((N - A_SPLIT, D_IN), jnp.float32),
        mesh=plsc.VectorSubcoreMesh(core_axis_name="c", subcore_axis_name="s"),
        scratch_types=[
            pltpu.VMEM((2, BLK, DEG, D_IN), jnp.float32),
            pltpu.VMEM((2, BLK, D_IN), jnp.float32),
            pltpu.SemaphoreType.DMA,
            pltpu.SemaphoreType.DMA,
            pltpu.SemaphoreType.DMA,
            pltpu.SemaphoreType.DMA,
        ],
    )


def _tree_sum(vs):
    while len(vs) > 1:
        vs = [vs[a] + vs[a + 1] for a in range(0, len(vs) - 1, 2)] + (
            [vs[-1]] if len(vs) % 2 else [])
    return vs[0]


BN = 1000


def _tc_fused_body(self_ref, neib_ref, ws_ref, wn_ref, out_ref):
    s = _tree_sum([neib_ref[:, k, :] for k in range(DEG)])
    from_self = jnp.dot(self_ref[...], ws_ref[...],
                        preferred_element_type=jnp.float32)
    from_neibs = jnp.dot(s * (1.0 / DEG), wn_ref[...],
                         preferred_element_type=jnp.float32)
    out_ref[...] = jnp.maximum(from_self + from_neibs, 0.0)


_tc_fused = pl.pallas_call(
    _tc_fused_body,
    grid=(A_SPLIT // BN,),
    in_specs=[
        pl.BlockSpec((BN, D_IN), lambda i: (i, 0)),
        pl.BlockSpec((BN, DEG, D_IN), lambda i: (i, 0, 0)),
        pl.BlockSpec((D_IN, D_OUT), lambda i: (0, 0)),
        pl.BlockSpec((D_IN, D_OUT), lambda i: (0, 0)),
    ],
    out_specs=pl.BlockSpec((BN, D_OUT), lambda i: (i, 0)),
    out_shape=jax.ShapeDtypeStruct((N, D_OUT), jnp.float32),
)

OFF = A_SPLIT // BN


def _tc_combine_body(partial_ref, self_ref, sums_ref, ws_ref, wn_ref,
                     out_ref):
    del partial_ref
    from_self = jnp.dot(self_ref[...], ws_ref[...],
                        preferred_element_type=jnp.float32)
    from_neibs = jnp.dot(sums_ref[...] * (1.0 / DEG), wn_ref[...],
                         preferred_element_type=jnp.float32)
    out_ref[...] = jnp.maximum(from_self + from_neibs, 0.0)


_tc_combine = pl.pallas_call(
    _tc_combine_body,
    grid=((N - A_SPLIT) // BN,),
    in_specs=[
        pl.BlockSpec(memory_space=pl.ANY),
        pl.BlockSpec((BN, D_IN), lambda i: (i, 0)),
        pl.BlockSpec((BN, D_IN), lambda i: (i, 0)),
        pl.BlockSpec((D_IN, D_OUT), lambda i: (0, 0)),
        pl.BlockSpec((D_IN, D_OUT), lambda i: (0, 0)),
    ],
    out_specs=pl.BlockSpec((BN, D_OUT), lambda i: (i + OFF, 0)),
    out_shape=jax.ShapeDtypeStruct((N, D_OUT), jnp.float32),
    input_output_aliases={0: 0},
)


_tc_fused_all = pl.pallas_call(
    _tc_fused_body,
    grid=(N // BN,),
    in_specs=[
        pl.BlockSpec((BN, D_IN), lambda i: (i, 0)),
        pl.BlockSpec((BN, DEG, D_IN), lambda i: (i, 0, 0)),
        pl.BlockSpec((D_IN, D_OUT), lambda i: (0, 0)),
        pl.BlockSpec((D_IN, D_OUT), lambda i: (0, 0)),
    ],
    out_specs=pl.BlockSpec((BN, D_OUT), lambda i: (i, 0)),
    out_shape=jax.ShapeDtypeStruct((N, D_OUT), jnp.float32),
)


def kernel(self_vecs, neib_vecs, neib_weights, self_weights):
    sums_tail = _make_sc_neib_sum()(neib_vecs)
    out_a = _tc_fused(self_vecs, neib_vecs, self_weights, neib_weights)
    self_tail = lax.slice(self_vecs, (A_SPLIT, 0), (N, D_IN))
    return _tc_combine(out_a, self_tail, sums_tail, self_weights,
                       neib_weights)


# speedup vs baseline: 1.0443x; 1.0443x over previous
import functools

import jax
import jax.numpy as jnp
from jax import lax
from jax.experimental import pallas as pl
from jax.experimental.pallas import tpu as pltpu
from jax.experimental.pallas import tpu_sc as plsc

N = 10000
DEG = 16
D_IN = 256
D_OUT = 256

LANES = 16
NC, NS = 2, 16
NW = NC * NS
BLK = 8
NBLK = N // BLK
DJ = D_IN // LANES

A_SPLIT = 9000
B0 = A_SPLIT // BLK


def _sc_sum_body(neib_hbm, out_hbm, buf, acc, sem_in0, sem_in1, sem_out0,
                 sem_out1):
    wid = lax.axis_index("s") * NC + lax.axis_index("c")
    nt = (NBLK - B0 - 1 - wid) // NW + 1
    sems_in = (sem_in0, sem_in1)
    sems_out = (sem_out0, sem_out1)

    def start_in(t, slot):
        blk = (B0 + wid + t * NW) * BLK
        pltpu.async_copy(neib_hbm.at[pl.ds(blk, BLK)], buf.at[slot],
                         sems_in[slot])

    @pl.when(nt > 0)
    def _prime():
        start_in(0, 0)

    @pl.loop(0, nt, step=2)
    def _pair(t0):
        for p in range(2):
            t = t0 + p

            @pl.when(t < nt)
            def _one():
                pltpu.make_async_copy(neib_hbm.at[pl.ds(0, BLK)], buf.at[p],
                                      sems_in[p]).wait()

                @pl.when(t + 1 < nt)
                def _next():
                    start_in(t + 1, 1 - p)

                @pl.when(t >= 2)
                def _drain():
                    pltpu.make_async_copy(acc.at[p],
                                          out_hbm.at[pl.ds(0, BLK)],
                                          sems_out[p]).wait()

                @pl.loop(0, BLK)
                def _node(i):
                    for j in range(DJ):
                        vs = [buf[p, i, k, pl.ds(j * LANES, LANES)]
                              for k in range(DEG)]
                        while len(vs) > 1:
                            vs = [vs[a] + vs[a + 1]
                                  for a in range(0, len(vs) - 1, 2)] + (
                                      [vs[-1]] if len(vs) % 2 else [])
                        acc[p, i, pl.ds(j * LANES, LANES)] = vs[0]

                pltpu.async_copy(
                    acc.at[p],
                    out_hbm.at[pl.ds((wid + t * NW) * BLK, BLK)],
                    sems_out[p])

    parity = (nt - 1) % 2
    for p in range(2):
        @pl.when(jnp.logical_and(nt >= 1, parity == p))
        def _last():
            pltpu.make_async_copy(acc.at[p], out_hbm.at[pl.ds(0, BLK)],
                                  sems_out[p]).wait()

        @pl.when(jnp.logical_and(nt >= 2, parity == 1 - p))
        def _second_last():
            pltpu.make_async_copy(acc.at[p], out_hbm.at[pl.ds(0, BLK)],
                                  sems_out[p]).wait()


@functools.cache
def _make_sc_neib_sum():
    return pl.kernel(
        _sc_sum_body,
        out_type=jax.ShapeDtypeStruct((N - A_SPLIT, D_IN), jnp.float32),
        mesh=plsc.VectorSubcoreMesh(core_axis_name="c", subcore_axis_name="s"),
        scratch_types=[
            pltpu.VMEM((2, BLK, DEG, D_IN), jnp.float32),
            pltpu.VMEM((2, BLK, D_IN), jnp.float32),
            pltpu.SemaphoreType.DMA,
            pltpu.SemaphoreType.DMA,
            pltpu.SemaphoreType.DMA,
            pltpu.SemaphoreType.DMA,
        ],
    )


def _tree_sum(vs):
    while len(vs) > 1:
        vs = [vs[a] + vs[a + 1] for a in range(0, len(vs) - 1, 2)] + (
            [vs[-1]] if len(vs) % 2 else [])
    return vs[0]


BN = 1000


def _tc_fused_body(self_ref, neib_ref, ws_ref, wn_ref, out_ref):
    s = _tree_sum([neib_ref[:, k, :] for k in range(DEG)])
    from_self = jnp.dot(self_ref[...], ws_ref[...],
                        preferred_element_type=jnp.float32)
    from_neibs = jnp.dot(s * (1.0 / DEG), wn_ref[...],
                         preferred_element_type=jnp.float32)
    out_ref[...] = jnp.maximum(from_self + from_neibs, 0.0)


_tc_fused = pl.pallas_call(
    _tc_fused_body,
    grid=(A_SPLIT // BN,),
    in_specs=[
        pl.BlockSpec((BN, D_IN), lambda i: (i, 0)),
        pl.BlockSpec((BN, DEG, D_IN), lambda i: (i, 0, 0)),
        pl.BlockSpec((D_IN, D_OUT), lambda i: (0, 0)),
        pl.BlockSpec((D_IN, D_OUT), lambda i: (0, 0)),
    ],
    out_specs=pl.BlockSpec((BN, D_OUT), lambda i: (i, 0)),
    out_shape=jax.ShapeDtypeStruct((N, D_OUT), jnp.float32),
)

OFF = A_SPLIT // BN


def _tc_combine_body(partial_ref, self_ref, sums_ref, ws_ref, wn_ref,
                     out_ref):
    del partial_ref
    from_self = jnp.dot(self_ref[...], ws_ref[...],
                        preferred_element_type=jnp.float32)
    from_neibs = jnp.dot(sums_ref[...] * (1.0 / DEG), wn_ref[...],
                         preferred_element_type=jnp.float32)
    out_ref[...] = jnp.maximum(from_self + from_neibs, 0.0)


_tc_combine = pl.pallas_call(
    _tc_combine_body,
    grid=((N - A_SPLIT) // BN,),
    in_specs=[
        pl.BlockSpec(memory_space=pl.ANY),
        pl.BlockSpec((BN, D_IN), lambda i: (i, 0)),
        pl.BlockSpec((BN, D_IN), lambda i: (i, 0)),
        pl.BlockSpec((D_IN, D_OUT), lambda i: (0, 0)),
        pl.BlockSpec((D_IN, D_OUT), lambda i: (0, 0)),
    ],
    out_specs=pl.BlockSpec((BN, D_OUT), lambda i: (i + OFF, 0)),
    out_shape=jax.ShapeDtypeStruct((N, D_OUT), jnp.float32),
    input_output_aliases={0: 0},
)


_tc_fused_all = pl.pallas_call(
    _tc_fused_body,
    grid=(N // BN,),
    in_specs=[
        pl.BlockSpec((BN, D_IN), lambda i: (i, 0)),
        pl.BlockSpec((BN, DEG, D_IN), lambda i: (i, 0, 0)),
        pl.BlockSpec((D_IN, D_OUT), lambda i: (0, 0)),
        pl.BlockSpec((D_IN, D_OUT), lambda i: (0, 0)),
    ],
    out_specs=pl.BlockSpec((BN, D_OUT), lambda i: (i, 0)),
    out_shape=jax.ShapeDtypeStruct((N, D_OUT), jnp.float32),
)


def kernel(self_vecs, neib_vecs, neib_weights, self_weights):
    sums_tail = _make_sc_neib_sum()(neib_vecs)
    out_a = _tc_fused(self_vecs, neib_vecs, self_weights, neib_weights)
    self_tail = lax.slice(self_vecs, (A_SPLIT, 0), (N, D_IN))
    return _tc_combine(out_a, self_tail, sums_tail, self_weights,
                       neib_weights)


# speedup vs baseline: 1.3872x; 1.3283x over previous
import functools

import jax
import jax.numpy as jnp
from jax import lax
from jax.experimental import pallas as pl
from jax.experimental.pallas import tpu as pltpu
from jax.experimental.pallas import tpu_sc as plsc

N = 10000
DEG = 16
D_IN = 256
D_OUT = 256

LANES = 16
NC, NS = 2, 16
NW = NC * NS
BLK = 8
NBLK = N // BLK
DJ = D_IN // LANES

A_SPLIT = 9000
B0 = A_SPLIT // BLK


def _sc_sum_body(neib_hbm, out_hbm, buf, acc, sem_in0, sem_in1, sem_out0,
                 sem_out1):
    wid = lax.axis_index("s") * NC + lax.axis_index("c")
    nt = (NBLK - B0 - 1 - wid) // NW + 1
    sems_in = (sem_in0, sem_in1)
    sems_out = (sem_out0, sem_out1)

    def start_in(t, slot):
        blk = (B0 + wid + t * NW) * BLK
        pltpu.async_copy(neib_hbm.at[pl.ds(blk, BLK)], buf.at[slot],
                         sems_in[slot])

    @pl.when(nt > 0)
    def _prime():
        start_in(0, 0)

    @pl.loop(0, nt, step=2)
    def _pair(t0):
        for p in range(2):
            t = t0 + p

            @pl.when(t < nt)
            def _one():
                pltpu.make_async_copy(neib_hbm.at[pl.ds(0, BLK)], buf.at[p],
                                      sems_in[p]).wait()

                @pl.when(t + 1 < nt)
                def _next():
                    start_in(t + 1, 1 - p)

                @pl.when(t >= 2)
                def _drain():
                    pltpu.make_async_copy(acc.at[p],
                                          out_hbm.at[pl.ds(0, BLK)],
                                          sems_out[p]).wait()

                @pl.loop(0, BLK)
                def _node(i):
                    for j in range(DJ):
                        vs = [buf[p, i, k, pl.ds(j * LANES, LANES)]
                              for k in range(DEG)]
                        while len(vs) > 1:
                            vs = [vs[a] + vs[a + 1]
                                  for a in range(0, len(vs) - 1, 2)] + (
                                      [vs[-1]] if len(vs) % 2 else [])
                        acc[p, i, pl.ds(j * LANES, LANES)] = vs[0]

                pltpu.async_copy(
                    acc.at[p],
                    out_hbm.at[pl.ds((wid + t * NW) * BLK, BLK)],
                    sems_out[p])

    parity = (nt - 1) % 2
    for p in range(2):
        @pl.when(jnp.logical_and(nt >= 1, parity == p))
        def _last():
            pltpu.make_async_copy(acc.at[p], out_hbm.at[pl.ds(0, BLK)],
                                  sems_out[p]).wait()

        @pl.when(jnp.logical_and(nt >= 2, parity == 1 - p))
        def _second_last():
            pltpu.make_async_copy(acc.at[p], out_hbm.at[pl.ds(0, BLK)],
                                  sems_out[p]).wait()


@functools.cache
def _make_sc_neib_sum():
    return pl.kernel(
        _sc_sum_body,
        out_type=jax.ShapeDtypeStruct((N - A_SPLIT, D_IN), jnp.float32),
        mesh=plsc.VectorSubcoreMesh(core_axis_name="c", subcore_axis_name="s"),
        scratch_types=[
            pltpu.VMEM((2, BLK, DEG, D_IN), jnp.float32),
            pltpu.VMEM((2, BLK, D_IN), jnp.float32),
            pltpu.SemaphoreType.DMA,
            pltpu.SemaphoreType.DMA,
            pltpu.SemaphoreType.DMA,
            pltpu.SemaphoreType.DMA,
        ],
    )


def _tree_sum(vs):
    while len(vs) > 1:
        vs = [vs[a] + vs[a + 1] for a in range(0, len(vs) - 1, 2)] + (
            [vs[-1]] if len(vs) % 2 else [])
    return vs[0]


BN = 1000


def _tc_fused_body(self_ref, neib_ref, ws_ref, wn_ref, out_ref):
    s = _tree_sum([neib_ref[:, k, :] for k in range(DEG)])
    from_self = jnp.dot(self_ref[...], ws_ref[...],
                        preferred_element_type=jnp.float32)
    from_neibs = jnp.dot(s * (1.0 / DEG), wn_ref[...],
                         preferred_element_type=jnp.float32)
    out_ref[...] = jnp.maximum(from_self + from_neibs, 0.0)


_tc_fused = pl.pallas_call(
    _tc_fused_body,
    grid=(A_SPLIT // BN,),
    in_specs=[
        pl.BlockSpec((BN, D_IN), lambda i: (i, 0)),
        pl.BlockSpec((BN, DEG, D_IN), lambda i: (i, 0, 0)),
        pl.BlockSpec((D_IN, D_OUT), lambda i: (0, 0)),
        pl.BlockSpec((D_IN, D_OUT), lambda i: (0, 0)),
    ],
    out_specs=pl.BlockSpec((BN, D_OUT), lambda i: (i, 0)),
    out_shape=jax.ShapeDtypeStruct((N, D_OUT), jnp.float32),
)

OFF = A_SPLIT // BN


def _tc_combine_body(partial_ref, self_ref, sums_ref, ws_ref, wn_ref,
                     out_ref):
    del partial_ref
    from_self = jnp.dot(self_ref[...], ws_ref[...],
                        preferred_element_type=jnp.float32)
    from_neibs = jnp.dot(sums_ref[...] * (1.0 / DEG), wn_ref[...],
                         preferred_element_type=jnp.float32)
    out_ref[...] = jnp.maximum(from_self + from_neibs, 0.0)


_tc_combine = pl.pallas_call(
    _tc_combine_body,
    grid=((N - A_SPLIT) // BN,),
    in_specs=[
        pl.BlockSpec(memory_space=pl.ANY),
        pl.BlockSpec((BN, D_IN), lambda i: (i, 0)),
        pl.BlockSpec((BN, D_IN), lambda i: (i, 0)),
        pl.BlockSpec((D_IN, D_OUT), lambda i: (0, 0)),
        pl.BlockSpec((D_IN, D_OUT), lambda i: (0, 0)),
    ],
    out_specs=pl.BlockSpec((BN, D_OUT), lambda i: (i + OFF, 0)),
    out_shape=jax.ShapeDtypeStruct((N, D_OUT), jnp.float32),
    input_output_aliases={0: 0},
)


_tc_fused_all = pl.pallas_call(
    _tc_fused_body,
    grid=(N // BN,),
    in_specs=[
        pl.BlockSpec((BN, D_IN), lambda i: (i, 0)),
        pl.BlockSpec((BN, DEG, D_IN), lambda i: (i, 0, 0)),
        pl.BlockSpec((D_IN, D_OUT), lambda i: (0, 0)),
        pl.BlockSpec((D_IN, D_OUT), lambda i: (0, 0)),
    ],
    out_specs=pl.BlockSpec((BN, D_OUT), lambda i: (i, 0)),
    out_shape=jax.ShapeDtypeStruct((N, D_OUT), jnp.float32),
)


def kernel(self_vecs, neib_vecs, neib_weights, self_weights):
    return _tc_fused_all(self_vecs, neib_vecs, self_weights, neib_weights)
